# Initial kernel scaffold; baseline (speedup 1.0000x reference)
import jax
import jax.numpy as jnp
from jax.experimental import pallas as pl


def kernel(x, table):
    raise NotImplementedError("write your pallas kernel here")


# speedup vs baseline: 1.6837x; 1.6837x over previous
import jax
import jax.numpy as jnp
from jax import lax
from jax.experimental import pallas as pl
from jax.experimental.pallas import tpu as pltpu
from jax.experimental.pallas import tpu_sc as plsc

VOCAB = 1000000
EMB_DIM = 64
NC = 2
NS = 16
NW = NC * NS

CHUNK = 128


def _emb_kernel_body(x_hbm, table_hbm, out_hbm, idx_v, rows_v, sem):
    wid = lax.axis_index("s") * NC + lax.axis_index("c")
    n_rows_per_w = idx_v.shape[0]
    base = wid * (n_rows_per_w * CHUNK)

    pltpu.sync_copy(x_hbm.at[wid], idx_v)

    def body(j, carry):
        pltpu.async_copy(table_hbm.at[idx_v.at[j]], rows_v, sem).wait()
        pltpu.sync_copy(rows_v, out_hbm.at[pl.ds(base + j * CHUNK, CHUNK)])
        return carry

    lax.fori_loop(0, n_rows_per_w, body, 0)


def kernel(x, table):
    B = x.shape[0] * x.shape[1]
    assert B % (NW * CHUNK) == 0
    n_rows_per_w = B // (NW * CHUNK)
    x_flat = x.reshape(NW, n_rows_per_w, CHUNK).astype(jnp.int32)

    mesh = plsc.VectorSubcoreMesh(core_axis_name="c", subcore_axis_name="s")
    out = pl.kernel(
        _emb_kernel_body,
        out_type=jax.ShapeDtypeStruct((B, EMB_DIM), jnp.float32),
        mesh=mesh,
        scratch_types=[
            pltpu.VMEM((n_rows_per_w, CHUNK), jnp.int32),
            pltpu.VMEM((CHUNK, EMB_DIM), jnp.float32),
            pltpu.SemaphoreType.DMA,
        ],
        compiler_params=pltpu.CompilerParams(use_tc_tiling_on_sc=False),
    )(x_flat, table)
    return out.reshape(x.shape[0], x.shape[1], EMB_DIM)


# speedup vs baseline: 1.8765x; 1.1145x over previous
import jax
import jax.numpy as jnp
from jax import lax
from jax.experimental import pallas as pl
from jax.experimental.pallas import tpu as pltpu
from jax.experimental.pallas import tpu_sc as plsc

VOCAB = 1000000
EMB_DIM = 64
NC = 2
NS = 16
NW = NC * NS

CHUNK = 128
K = 4
GROUP = K * CHUNK


def _emb_kernel_body(x_hbm, table_hbm, out_hbm, idx_v, buf0, buf1, gsem0,
                     gsem1, ssem0, ssem1):
    wid = lax.axis_index("s") * NC + lax.axis_index("c")
    n_idx_rows = idx_v.shape[0]
    n_groups = n_idx_rows // K
    base = wid * (n_idx_rows * CHUNK)

    pltpu.sync_copy(x_hbm.at[wid], idx_v)

    def fire_gathers(g, buf, gsem):
        for j in range(K):
            pltpu.async_copy(table_hbm.at[idx_v.at[g * K + j]],
                             buf.at[pl.ds(j * CHUNK, CHUNK)], gsem)

    def wait_gathers(buf, gsem):
        for j in range(K):
            pltpu.make_async_copy(out_hbm.at[pl.ds(0, CHUNK)],
                                  buf.at[pl.ds(j * CHUNK, CHUNK)], gsem).wait()

    def fire_store(g, buf, ssem):
        pltpu.async_copy(buf, out_hbm.at[pl.ds(base + g * GROUP, GROUP)], ssem)

    def wait_store(buf, ssem):
        pltpu.make_async_copy(buf, out_hbm.at[pl.ds(base, GROUP)], ssem).wait()

    fire_gathers(0, buf0, gsem0)
    fire_gathers(1, buf1, gsem1)
    wait_gathers(buf0, gsem0)
    fire_store(0, buf0, ssem0)

    def body(i, carry):
        g = 2 * i + 2
        wait_store(buf0, ssem0)
        fire_gathers(g, buf0, gsem0)
        wait_gathers(buf1, gsem1)
        fire_store(g - 1, buf1, ssem1)
        wait_store(buf1, ssem1)
        fire_gathers(g + 1, buf1, gsem1)
        wait_gathers(buf0, gsem0)
        fire_store(g, buf0, ssem0)
        return carry

    lax.fori_loop(0, (n_groups - 2) // 2, body, 0)

    wait_gathers(buf1, gsem1)
    fire_store(n_groups - 1, buf1, ssem1)
    wait_store(buf0, ssem0)
    wait_store(buf1, ssem1)


def kernel(x, table):
    B = x.shape[0] * x.shape[1]
    assert B % (NW * GROUP * 2) == 0
    n_idx_rows = B // (NW * CHUNK)
    x_flat = x.reshape(NW, n_idx_rows, CHUNK).astype(jnp.int32)

    mesh = plsc.VectorSubcoreMesh(core_axis_name="c", subcore_axis_name="s")
    out = pl.kernel(
        _emb_kernel_body,
        out_type=jax.ShapeDtypeStruct((B, EMB_DIM), jnp.float32),
        mesh=mesh,
        scratch_types=[
            pltpu.VMEM((n_idx_rows, CHUNK), jnp.int32),
            pltpu.VMEM((GROUP, EMB_DIM), jnp.float32),
            pltpu.VMEM((GROUP, EMB_DIM), jnp.float32),
            pltpu.SemaphoreType.DMA,
            pltpu.SemaphoreType.DMA,
            pltpu.SemaphoreType.DMA,
            pltpu.SemaphoreType.DMA,
        ],
        compiler_params=pltpu.CompilerParams(use_tc_tiling_on_sc=False),
    )(x_flat, table)
    return out.reshape(x.shape[0], x.shape[1], EMB_DIM)
